# baseline (speedup 1.0000x reference)
import functools

import jax
import jax.numpy as jnp
from jax import lax
from jax.experimental import pallas as pl
from jax.experimental.pallas import tpu as pltpu
from jax.experimental.pallas import tpu_sc as plsc

_VOCAB = 100000
_EMBED = 32
_HIDDEN = 128
_CTX = 200
_VB = 4096
_NB = -(-_VOCAB // _VB)


def _make_embed_sum():
    mesh = plsc.VectorSubcoreMesh(core_axis_name="c", subcore_axis_name="s")

    @functools.partial(
        pl.kernel,
        mesh=mesh,
        compiler_params=pltpu.CompilerParams(use_tc_tiling_on_sc=False),
        out_type=jax.ShapeDtypeStruct((_EMBED,), jnp.float32),
        scratch_types=[
            pltpu.VMEM((_CTX,), jnp.int32),
            pltpu.VMEM((_CTX, _EMBED), jnp.float32),
            pltpu.VMEM((_EMBED,), jnp.float32),
            pltpu.SemaphoreType.DMA,
        ],
    )
    def embed_sum(idx_hbm, table_hbm, out_hbm, idx_v, rows_v, acc_v, sem):
        @pl.when((lax.axis_index("c") == 0) & (lax.axis_index("s") == 0))
        def _():
            pltpu.sync_copy(idx_hbm, idx_v)
            pltpu.async_copy(table_hbm.at[idx_v], rows_v, sem).wait()

            def body(i, carry):
                a0, a1 = carry
                return (a0 + rows_v[i, pl.ds(0, 16)],
                        a1 + rows_v[i, pl.ds(16, 16)])

            z = jnp.zeros((16,), jnp.float32)
            a0, a1 = lax.fori_loop(0, _CTX, body, (z, z))
            acc_v[pl.ds(0, 16)] = a0
            acc_v[pl.ds(16, 16)] = a1
            pltpu.sync_copy(acc_v, out_hbm)

    return embed_sum


def _logits_body(e_ref, w1t_ref, b1_ref, w2_ref, b2_ref, lg_ref, lz_ref,
                 h_s, m_s, s_s):
    i = pl.program_id(0)

    @pl.when(i == 0)
    def _():
        h = jnp.dot(e_ref[...], w1t_ref[...],
                    preferred_element_type=jnp.float32) + b1_ref[...]
        h_s[...] = jnp.maximum(h, 0.0)
        m_s[0] = -jnp.inf
        s_s[0] = 0.0

    h = h_s[...]
    logits = lax.dot_general(
        h, w2_ref[...], (((1,), (1,)), ((), ())),
        preferred_element_type=jnp.float32) + b2_ref[...]
    lg_ref[...] = logits

    col = lax.broadcasted_iota(jnp.int32, (1, _VB), 1) + i * _VB
    lm = jnp.where(col < _VOCAB, logits, -jnp.inf)
    bmax = jnp.max(lm)
    m_old = m_s[0]
    m_new = jnp.maximum(m_old, bmax)
    s_s[0] = s_s[0] * jnp.exp(m_old - m_new) + jnp.sum(jnp.exp(lm - m_new))
    m_s[0] = m_new

    @pl.when(i == pl.num_programs(0) - 1)
    def _():
        lz_ref[...] = jnp.full((1, 1), m_s[0] + jnp.log(s_s[0]),
                               dtype=jnp.float32)


def _normalize_body(lg_ref, lz_ref, out_ref):
    out_ref[...] = lg_ref[...] - lz_ref[0, 0]


def kernel(inputs, emb_table, W1, b1, W2, b2):
    e = _make_embed_sum()(inputs, emb_table).reshape(1, _EMBED)

    w1t = W1.T
    b1r = b1.reshape(1, _HIDDEN)
    b2r = b2.reshape(1, _VOCAB)

    logits, logz = pl.pallas_call(
        _logits_body,
        grid=(_NB,),
        in_specs=[
            pl.BlockSpec((1, _EMBED), lambda i: (0, 0)),
            pl.BlockSpec((_EMBED, _HIDDEN), lambda i: (0, 0)),
            pl.BlockSpec((1, _HIDDEN), lambda i: (0, 0)),
            pl.BlockSpec((_VB, _HIDDEN), lambda i: (i, 0)),
            pl.BlockSpec((1, _VB), lambda i: (0, i)),
        ],
        out_specs=[
            pl.BlockSpec((1, _VB), lambda i: (0, i)),
            pl.BlockSpec((1, 1), lambda i: (0, 0)),
        ],
        out_shape=[
            jax.ShapeDtypeStruct((1, _VOCAB), jnp.float32),
            jax.ShapeDtypeStruct((1, 1), jnp.float32),
        ],
        scratch_shapes=[
            pltpu.VMEM((1, _HIDDEN), jnp.float32),
            pltpu.SMEM((1,), jnp.float32),
            pltpu.SMEM((1,), jnp.float32),
        ],
    )(e, w1t, b1r, W2, b2r)

    log_probs = pl.pallas_call(
        _normalize_body,
        grid=(_NB,),
        in_specs=[
            pl.BlockSpec((1, _VB), lambda i: (0, i)),
            pl.BlockSpec(memory_space=pltpu.SMEM),
        ],
        out_specs=pl.BlockSpec((1, _VB), lambda i: (0, i)),
        out_shape=jax.ShapeDtypeStruct((1, _VOCAB), jnp.float32),
    )(logits, logz)

    return log_probs


# speedup vs baseline: 1.2282x; 1.2282x over previous
import functools

import jax
import jax.numpy as jnp
from jax import lax
from jax.experimental import pallas as pl
from jax.experimental.pallas import tpu as pltpu
from jax.experimental.pallas import tpu_sc as plsc

_VOCAB = 100000
_EMBED = 32
_HIDDEN = 128
_CTX = 200
_VB = 4096
_NB = -(-_VOCAB // _VB)


def _make_embed_sum():
    mesh = plsc.VectorSubcoreMesh(core_axis_name="c", subcore_axis_name="s")

    @functools.partial(
        pl.kernel,
        mesh=mesh,
        compiler_params=pltpu.CompilerParams(use_tc_tiling_on_sc=False),
        out_type=jax.ShapeDtypeStruct((_EMBED,), jnp.float32),
        scratch_types=[
            pltpu.VMEM((_CTX,), jnp.int32),
            pltpu.VMEM((_CTX, _EMBED), jnp.float32),
            pltpu.VMEM((_EMBED,), jnp.float32),
            pltpu.SemaphoreType.DMA,
        ],
    )
    def embed_sum(idx_hbm, table_hbm, out_hbm, idx_v, rows_v, acc_v, sem):
        @pl.when((lax.axis_index("c") == 0) & (lax.axis_index("s") == 0))
        def _():
            pltpu.sync_copy(idx_hbm, idx_v)
            pltpu.async_copy(table_hbm.at[idx_v], rows_v, sem).wait()

            def body(i, carry):
                a0, a1 = carry
                return (a0 + rows_v[i, pl.ds(0, 16)],
                        a1 + rows_v[i, pl.ds(16, 16)])

            z = jnp.zeros((16,), jnp.float32)
            a0, a1 = lax.fori_loop(0, _CTX, body, (z, z))
            acc_v[pl.ds(0, 16)] = a0
            acc_v[pl.ds(16, 16)] = a1
            pltpu.sync_copy(acc_v, out_hbm)

    return embed_sum


def _logits_body(e_ref, w1t_ref, b1_ref, w2_ref, b2_ref, lg_ref, lz_ref,
                 h_s, m_s, s_s):
    i = pl.program_id(0)

    @pl.when(i == 0)
    def _():
        h = jnp.dot(e_ref[...], w1t_ref[...],
                    preferred_element_type=jnp.float32) + b1_ref[...]
        h_s[...] = jnp.maximum(h, 0.0)
        m_s[0] = -jnp.inf
        s_s[0] = 0.0

    h = h_s[...]
    logits = lax.dot_general(
        h, w2_ref[...], (((1,), (1,)), ((), ())),
        preferred_element_type=jnp.float32) + b2_ref[...]
    lg_ref[...] = logits

    col = lax.broadcasted_iota(jnp.int32, (1, _VB), 1) + i * _VB
    lm = jnp.where(col < _VOCAB, logits, -jnp.inf)
    bmax = jnp.max(lm)
    m_old = m_s[0]
    m_new = jnp.maximum(m_old, bmax)
    s_s[0] = s_s[0] * jnp.exp(m_old - m_new) + jnp.sum(jnp.exp(lm - m_new))
    m_s[0] = m_new

    @pl.when(i == pl.num_programs(0) - 1)
    def _():
        lz_ref[...] = jnp.full((1, 1), m_s[0] + jnp.log(s_s[0]),
                               dtype=jnp.float32)


def _normalize_body(lg_ref, lz_ref, out_ref):
    out_ref[...] = lg_ref[...] - lz_ref[0, 0]


def kernel(inputs, emb_table, W1, b1, W2, b2):
    e = jnp.sum(jnp.take(emb_table, inputs, axis=0), axis=0).reshape(1, _EMBED)

    w1t = W1.T
    b1r = b1.reshape(1, _HIDDEN)
    b2r = b2.reshape(1, _VOCAB)

    logits, logz = pl.pallas_call(
        _logits_body,
        grid=(_NB,),
        in_specs=[
            pl.BlockSpec((1, _EMBED), lambda i: (0, 0)),
            pl.BlockSpec((_EMBED, _HIDDEN), lambda i: (0, 0)),
            pl.BlockSpec((1, _HIDDEN), lambda i: (0, 0)),
            pl.BlockSpec((_VB, _HIDDEN), lambda i: (i, 0)),
            pl.BlockSpec((1, _VB), lambda i: (0, i)),
        ],
        out_specs=[
            pl.BlockSpec((1, _VB), lambda i: (0, i)),
            pl.BlockSpec((1, 1), lambda i: (0, 0)),
        ],
        out_shape=[
            jax.ShapeDtypeStruct((1, _VOCAB), jnp.float32),
            jax.ShapeDtypeStruct((1, 1), jnp.float32),
        ],
        scratch_shapes=[
            pltpu.VMEM((1, _HIDDEN), jnp.float32),
            pltpu.SMEM((1,), jnp.float32),
            pltpu.SMEM((1,), jnp.float32),
        ],
    )(e, w1t, b1r, W2, b2r)

    log_probs = pl.pallas_call(
        _normalize_body,
        grid=(_NB,),
        in_specs=[
            pl.BlockSpec((1, _VB), lambda i: (0, i)),
            pl.BlockSpec(memory_space=pltpu.SMEM),
        ],
        out_specs=pl.BlockSpec((1, _VB), lambda i: (0, i)),
        out_shape=jax.ShapeDtypeStruct((1, _VOCAB), jnp.float32),
    )(logits, logz)

    return log_probs


# speedup vs baseline: 1.4383x; 1.1711x over previous
import functools

import jax
import jax.numpy as jnp
from jax import lax
from jax.experimental import pallas as pl
from jax.experimental.pallas import tpu as pltpu
from jax.experimental.pallas import tpu_sc as plsc

_VOCAB = 100000
_EMBED = 32
_HIDDEN = 128
_CTX = 200
_VB = 8192
_NB = -(-_VOCAB // _VB)


def _make_embed_sum():
    mesh = plsc.VectorSubcoreMesh(core_axis_name="c", subcore_axis_name="s")

    @functools.partial(
        pl.kernel,
        mesh=mesh,
        compiler_params=pltpu.CompilerParams(use_tc_tiling_on_sc=False),
        out_type=jax.ShapeDtypeStruct((_EMBED,), jnp.float32),
        scratch_types=[
            pltpu.VMEM((_CTX,), jnp.int32),
            pltpu.VMEM((_CTX, _EMBED), jnp.float32),
            pltpu.VMEM((_EMBED,), jnp.float32),
            pltpu.SemaphoreType.DMA,
        ],
    )
    def embed_sum(idx_hbm, table_hbm, out_hbm, idx_v, rows_v, acc_v, sem):
        @pl.when((lax.axis_index("c") == 0) & (lax.axis_index("s") == 0))
        def _():
            pltpu.sync_copy(idx_hbm, idx_v)
            pltpu.async_copy(table_hbm.at[idx_v], rows_v, sem).wait()

            def body(i, carry):
                a0, a1 = carry
                return (a0 + rows_v[i, pl.ds(0, 16)],
                        a1 + rows_v[i, pl.ds(16, 16)])

            z = jnp.zeros((16,), jnp.float32)
            a0, a1 = lax.fori_loop(0, _CTX, body, (z, z))
            acc_v[pl.ds(0, 16)] = a0
            acc_v[pl.ds(16, 16)] = a1
            pltpu.sync_copy(acc_v, out_hbm)

    return embed_sum


def _logits_body(e_ref, w1t_ref, b1_ref, w2_ref, b2_ref, lg_ref, lz_ref,
                 h_s, m_s, s_s):
    i = pl.program_id(0)

    @pl.when(i == 0)
    def _():
        h = jnp.dot(e_ref[...], w1t_ref[...],
                    preferred_element_type=jnp.float32) + b1_ref[...]
        h_s[...] = jnp.maximum(h, 0.0)
        m_s[0] = -jnp.inf
        s_s[0] = 0.0

    h = h_s[...]
    logits = lax.dot_general(
        h, w2_ref[...], (((1,), (1,)), ((), ())),
        preferred_element_type=jnp.float32) + b2_ref[...]
    lg_ref[...] = logits

    col = lax.broadcasted_iota(jnp.int32, (1, _VB), 1) + i * _VB
    lm = jnp.where(col < _VOCAB, logits, -jnp.inf)
    bmax = jnp.max(lm)
    m_old = m_s[0]
    m_new = jnp.maximum(m_old, bmax)
    s_s[0] = s_s[0] * jnp.exp(m_old - m_new) + jnp.sum(jnp.exp(lm - m_new))
    m_s[0] = m_new

    @pl.when(i == pl.num_programs(0) - 1)
    def _():
        lz_ref[...] = jnp.full((1, 1), m_s[0] + jnp.log(s_s[0]),
                               dtype=jnp.float32)


def _normalize_body(lg_ref, lz_ref, out_ref):
    out_ref[...] = lg_ref[...] - lz_ref[0, 0]


def kernel(inputs, emb_table, W1, b1, W2, b2):
    e = jnp.sum(jnp.take(emb_table, inputs, axis=0), axis=0).reshape(1, _EMBED)

    w1t = W1.T
    b1r = b1.reshape(1, _HIDDEN)
    b2r = b2.reshape(1, _VOCAB)

    logits, logz = pl.pallas_call(
        _logits_body,
        grid=(_NB,),
        in_specs=[
            pl.BlockSpec((1, _EMBED), lambda i: (0, 0)),
            pl.BlockSpec((_EMBED, _HIDDEN), lambda i: (0, 0)),
            pl.BlockSpec((1, _HIDDEN), lambda i: (0, 0)),
            pl.BlockSpec((_VB, _HIDDEN), lambda i: (i, 0)),
            pl.BlockSpec((1, _VB), lambda i: (0, i)),
        ],
        out_specs=[
            pl.BlockSpec((1, _VB), lambda i: (0, i)),
            pl.BlockSpec((1, 1), lambda i: (0, 0)),
        ],
        out_shape=[
            jax.ShapeDtypeStruct((1, _VOCAB), jnp.float32),
            jax.ShapeDtypeStruct((1, 1), jnp.float32),
        ],
        scratch_shapes=[
            pltpu.VMEM((1, _HIDDEN), jnp.float32),
            pltpu.SMEM((1,), jnp.float32),
            pltpu.SMEM((1,), jnp.float32),
        ],
    )(e, w1t, b1r, W2, b2r)

    log_probs = pl.pallas_call(
        _normalize_body,
        grid=(_NB,),
        in_specs=[
            pl.BlockSpec((1, _VB), lambda i: (0, i)),
            pl.BlockSpec(memory_space=pltpu.SMEM),
        ],
        out_specs=pl.BlockSpec((1, _VB), lambda i: (0, i)),
        out_shape=jax.ShapeDtypeStruct((1, _VOCAB), jnp.float32),
    )(logits, logz)

    return log_probs


# speedup vs baseline: 1.5329x; 1.0657x over previous
import functools

import jax
import jax.numpy as jnp
from jax import lax
from jax.experimental import pallas as pl
from jax.experimental.pallas import tpu as pltpu
from jax.experimental.pallas import tpu_sc as plsc

_VOCAB = 100000
_EMBED = 32
_HIDDEN = 128
_CTX = 200
_VB = 16384
_NB = -(-_VOCAB // _VB)


def _make_embed_sum():
    mesh = plsc.VectorSubcoreMesh(core_axis_name="c", subcore_axis_name="s")

    @functools.partial(
        pl.kernel,
        mesh=mesh,
        compiler_params=pltpu.CompilerParams(use_tc_tiling_on_sc=False),
        out_type=jax.ShapeDtypeStruct((_EMBED,), jnp.float32),
        scratch_types=[
            pltpu.VMEM((_CTX,), jnp.int32),
            pltpu.VMEM((_CTX, _EMBED), jnp.float32),
            pltpu.VMEM((_EMBED,), jnp.float32),
            pltpu.SemaphoreType.DMA,
        ],
    )
    def embed_sum(idx_hbm, table_hbm, out_hbm, idx_v, rows_v, acc_v, sem):
        @pl.when((lax.axis_index("c") == 0) & (lax.axis_index("s") == 0))
        def _():
            pltpu.sync_copy(idx_hbm, idx_v)
            pltpu.async_copy(table_hbm.at[idx_v], rows_v, sem).wait()

            def body(i, carry):
                a0, a1 = carry
                return (a0 + rows_v[i, pl.ds(0, 16)],
                        a1 + rows_v[i, pl.ds(16, 16)])

            z = jnp.zeros((16,), jnp.float32)
            a0, a1 = lax.fori_loop(0, _CTX, body, (z, z))
            acc_v[pl.ds(0, 16)] = a0
            acc_v[pl.ds(16, 16)] = a1
            pltpu.sync_copy(acc_v, out_hbm)

    return embed_sum


def _logits_body(e_ref, w1t_ref, b1_ref, w2_ref, b2_ref, lg_ref, lz_ref,
                 h_s, m_s, s_s):
    i = pl.program_id(0)

    @pl.when(i == 0)
    def _():
        h = jnp.dot(e_ref[...], w1t_ref[...],
                    preferred_element_type=jnp.float32) + b1_ref[...]
        h_s[...] = jnp.maximum(h, 0.0)
        m_s[0] = -jnp.inf
        s_s[0] = 0.0

    h = h_s[...]
    logits = lax.dot_general(
        h, w2_ref[...], (((1,), (1,)), ((), ())),
        preferred_element_type=jnp.float32) + b2_ref[...]
    lg_ref[...] = logits

    col = lax.broadcasted_iota(jnp.int32, (1, _VB), 1) + i * _VB
    lm = jnp.where(col < _VOCAB, logits, -jnp.inf)
    bmax = jnp.max(lm)
    m_old = m_s[0]
    m_new = jnp.maximum(m_old, bmax)
    s_s[0] = s_s[0] * jnp.exp(m_old - m_new) + jnp.sum(jnp.exp(lm - m_new))
    m_s[0] = m_new

    @pl.when(i == pl.num_programs(0) - 1)
    def _():
        lz_ref[...] = jnp.full((1, 1), m_s[0] + jnp.log(s_s[0]),
                               dtype=jnp.float32)


def _normalize_body(lg_ref, lz_ref, out_ref):
    out_ref[...] = lg_ref[...] - lz_ref[0, 0]


def kernel(inputs, emb_table, W1, b1, W2, b2):
    e = jnp.sum(jnp.take(emb_table, inputs, axis=0), axis=0).reshape(1, _EMBED)

    w1t = W1.T
    b1r = b1.reshape(1, _HIDDEN)
    b2r = b2.reshape(1, _VOCAB)

    logits, logz = pl.pallas_call(
        _logits_body,
        grid=(_NB,),
        in_specs=[
            pl.BlockSpec((1, _EMBED), lambda i: (0, 0)),
            pl.BlockSpec((_EMBED, _HIDDEN), lambda i: (0, 0)),
            pl.BlockSpec((1, _HIDDEN), lambda i: (0, 0)),
            pl.BlockSpec((_VB, _HIDDEN), lambda i: (i, 0)),
            pl.BlockSpec((1, _VB), lambda i: (0, i)),
        ],
        out_specs=[
            pl.BlockSpec((1, _VB), lambda i: (0, i)),
            pl.BlockSpec((1, 1), lambda i: (0, 0)),
        ],
        out_shape=[
            jax.ShapeDtypeStruct((1, _VOCAB), jnp.float32),
            jax.ShapeDtypeStruct((1, 1), jnp.float32),
        ],
        scratch_shapes=[
            pltpu.VMEM((1, _HIDDEN), jnp.float32),
            pltpu.SMEM((1,), jnp.float32),
            pltpu.SMEM((1,), jnp.float32),
        ],
    )(e, w1t, b1r, W2, b2r)

    log_probs = pl.pallas_call(
        _normalize_body,
        grid=(_NB,),
        in_specs=[
            pl.BlockSpec((1, _VB), lambda i: (0, i)),
            pl.BlockSpec(memory_space=pltpu.SMEM),
        ],
        out_specs=pl.BlockSpec((1, _VB), lambda i: (0, i)),
        out_shape=jax.ShapeDtypeStruct((1, _VOCAB), jnp.float32),
    )(logits, logz)

    return log_probs
